# baseline (speedup 1.0000x reference)
import functools

import jax
import jax.numpy as jnp
from jax import lax
from jax.experimental import pallas as pl
from jax.experimental.pallas import tpu as pltpu
from jax.experimental.pallas import tpu_sc as plsc

EMBED_DIM = 64
VALUES_LEN = 81920
NUM_TABLES = 4
NUM_WORKERS = 32
WORKERS_PER_TABLE = NUM_WORKERS // NUM_TABLES
PER_W = VALUES_LEN // WORKERS_PER_TABLE
CHUNK = 128
NCHUNK = PER_W // CHUNK
NBUF = 5

_mesh = plsc.VectorSubcoreMesh(core_axis_name="c", subcore_axis_name="s")


@functools.partial(
    pl.kernel,
    mesh=_mesh,
    out_type=jax.ShapeDtypeStruct((NUM_TABLES * VALUES_LEN // 2,
                                   2 * EMBED_DIM), jnp.float32),
    scratch_types=[
        pltpu.VMEM((NBUF, CHUNK), jnp.int32),
        pltpu.VMEM((NBUF, CHUNK), jnp.int32),
        pltpu.VMEM((NBUF, CHUNK, 2 * EMBED_DIM), jnp.float32),
        pltpu.SemaphoreType.DMA((NBUF,)),
        pltpu.SemaphoreType.DMA((NBUF,)),
        pltpu.SemaphoreType.DMA((NBUF,)),
    ],
    compiler_params=pltpu.CompilerParams(needs_layout_passes=False),
)
def _grouped_embedding(v0, v1, v2, v3, w0, w1, w2, w3, out,
                       idx_v, hr_v, pair_v, sem_g, sem_w, sem_i):
    wid = lax.axis_index("s") * 2 + lax.axis_index("c")
    table_id = wid // WORKERS_PER_TABLE
    sub = wid % WORKERS_PER_TABLE
    iota16 = lax.iota(jnp.int32, 16)

    for t, (vals, table) in enumerate(
        ((v0, w0), (v1, w1), (v2, w2), (v3, w3))):

        @pl.when(table_id == t)
        def _(vals=vals, table=table, t=t):
            out_base = t * VALUES_LEN + sub * PER_W
            idx_base = sub * PER_W

            def idx_slice(c):
                start = pl.multiple_of(idx_base + c * CHUNK, CHUNK)
                return vals.at[pl.ds(start, CHUNK)]

            def out_slice(c):
                start = pl.multiple_of((out_base + c * CHUNK) // 2, CHUNK // 2)
                return out.at[pl.ds(start, CHUNK // 2), :]

            for b in range(NBUF):
                pltpu.async_copy(idx_slice(b), idx_v.at[b], sem_i.at[b])

            def compute_hr(b):
                def one(m, carry):
                    r = idx_v[b, pl.ds(m * 16, 16)]
                    hr_v[b, pl.ds(m * 16, 16)] = lax.shift_right_logical(r, 1)
                    return carry
                lax.fori_loop(0, CHUNK // 16, one, 0)

            def extract(b):
                half_iota = lax.shift_right_logical(iota16, 1)
                par_col = (iota16 & 1) * EMBED_DIM

                def per_group(g, carry):
                    rowv = g * 16 + iota16
                    dst_rowv = g * 8 + half_iota
                    sel = idx_v[b, pl.ds(g * 16, 16)] & 1
                    colbase = sel * EMBED_DIM

                    def per_col(q, carry2):
                        for u in range(4):
                            cc = q * 4 + u
                            val = plsc.load_gather(
                                pair_v.at[b], [rowv, colbase + cc])
                            plsc.store_scatter(
                                pair_v.at[b], [dst_rowv, par_col + cc], val)
                        return carry2

                    lax.fori_loop(0, EMBED_DIM // 4, per_col, 0)
                    return carry
                lax.fori_loop(0, CHUNK // 16, per_group, 0)

            def body(i, carry):
                for b in range(NBUF):
                    c = i * NBUF + b

                    @pl.when(i > 0)
                    def _(b=b, c=c):
                        pltpu.make_async_copy(
                            pair_v.at[b, pl.ds(0, CHUNK // 2), :],
                            out_slice(c), sem_w.at[b]).wait()

                    pltpu.make_async_copy(
                        idx_slice(c), idx_v.at[b], sem_i.at[b]).wait()
                    compute_hr(b)
                    pltpu.async_copy(
                        table.at[hr_v.at[b]], pair_v.at[b], sem_g.at[b])
                for b in range(NBUF):
                    c = i * NBUF + b
                    pltpu.make_async_copy(
                        table.at[hr_v.at[b]], pair_v.at[b], sem_g.at[b]).wait()
                    extract(b)
                    pltpu.async_copy(
                        pair_v.at[b, pl.ds(0, CHUNK // 2), :],
                        out_slice(c), sem_w.at[b])

                    @pl.when(c + NBUF < NCHUNK)
                    def _(b=b, c=c):
                        pltpu.async_copy(
                            idx_slice(c + NBUF), idx_v.at[b], sem_i.at[b])
                return carry

            lax.fori_loop(0, NCHUNK // NBUF, body, 0)
            for b in range(NBUF):
                pltpu.make_async_copy(
                    pair_v.at[b, pl.ds(0, CHUNK // 2), :],
                    out_slice(b), sem_w.at[b]).wait()


def kernel(values_0, values_1, values_2, values_3, W0, W1, W2, W3):
    wp = [w.reshape(w.shape[0] // 2, 2 * EMBED_DIM)
          for w in (W0, W1, W2, W3)]
    res = _grouped_embedding(values_0, values_1, values_2, values_3, *wp)
    return res.reshape(NUM_TABLES * VALUES_LEN, EMBED_DIM)


# speedup vs baseline: 1.2559x; 1.2559x over previous
import functools

import jax
import jax.numpy as jnp
from jax import lax
from jax.experimental import pallas as pl
from jax.experimental.pallas import tpu as pltpu
from jax.experimental.pallas import tpu_sc as plsc

EMBED_DIM = 64
VALUES_LEN = 81920
NUM_TABLES = 4
NUM_WORKERS = 32
WORKERS_PER_TABLE = NUM_WORKERS // NUM_TABLES
PER_W = VALUES_LEN // WORKERS_PER_TABLE
CHUNK = 128
NCHUNK = PER_W // CHUNK
NBUF = 4

_mesh = plsc.VectorSubcoreMesh(core_axis_name="c", subcore_axis_name="s")


@functools.partial(
    pl.kernel,
    mesh=_mesh,
    out_type=jax.ShapeDtypeStruct((EMBED_DIM, NUM_TABLES * VALUES_LEN),
                                  jnp.float32),
    scratch_types=[
        pltpu.VMEM((NBUF, CHUNK), jnp.int32),
        pltpu.VMEM((NBUF, CHUNK), jnp.int32),
        pltpu.VMEM((NBUF, CHUNK, 2 * EMBED_DIM), jnp.float32),
        pltpu.VMEM((2, EMBED_DIM, CHUNK), jnp.float32),
        pltpu.SemaphoreType.DMA((NBUF,)),
        pltpu.SemaphoreType.DMA((2,)),
        pltpu.SemaphoreType.DMA((NBUF,)),
    ],
    compiler_params=pltpu.CompilerParams(needs_layout_passes=False),
)
def _grouped_embedding(v0, v1, v2, v3, w0, w1, w2, w3, out,
                       idx_v, hr_v, pair_v, comp_t, sem_g, sem_w, sem_i):
    wid = lax.axis_index("s") * 2 + lax.axis_index("c")
    table_id = wid // WORKERS_PER_TABLE
    sub = wid % WORKERS_PER_TABLE
    iota16 = lax.iota(jnp.int32, 16)

    for t, (vals, table) in enumerate(
        ((v0, w0), (v1, w1), (v2, w2), (v3, w3))):

        @pl.when(table_id == t)
        def _(vals=vals, table=table, t=t):
            out_base = t * VALUES_LEN + sub * PER_W
            idx_base = sub * PER_W

            def idx_slice(c):
                start = pl.multiple_of(idx_base + c * CHUNK, CHUNK)
                return vals.at[pl.ds(start, CHUNK)]

            def out_slice(c):
                start = pl.multiple_of(out_base + c * CHUNK, CHUNK)
                return out.at[:, pl.ds(start, CHUNK)]

            for b in range(NBUF):
                pltpu.async_copy(idx_slice(b), idx_v.at[b], sem_i.at[b])

            def compute_hr(b):
                def one(m, carry):
                    r = idx_v[b, pl.ds(m * 16, 16)]
                    hr_v[b, pl.ds(m * 16, 16)] = lax.shift_right_logical(r, 1)
                    return carry
                lax.fori_loop(0, CHUNK // 16, one, 0)

            def extract(b, b2):
                def per_group(g, carry):
                    rowv = g * 16 + iota16
                    sel = idx_v[b, pl.ds(g * 16, 16)] & 1
                    colbase = sel * EMBED_DIM

                    def per_col(q, carry2):
                        for u in range(4):
                            cc = q * 4 + u
                            val = plsc.load_gather(
                                pair_v.at[b], [rowv, colbase + cc])
                            comp_t[b2, cc, pl.ds(g * 16, 16)] = val
                        return carry2

                    lax.fori_loop(0, EMBED_DIM // 4, per_col, 0)
                    return carry
                lax.fori_loop(0, CHUNK // 16, per_group, 0)

            def body(i, carry):
                for b in range(NBUF):
                    c = i * NBUF + b
                    pltpu.make_async_copy(
                        idx_slice(c), idx_v.at[b], sem_i.at[b]).wait()
                    compute_hr(b)
                    pltpu.async_copy(
                        table.at[hr_v.at[b]], pair_v.at[b], sem_g.at[b])
                for b in range(NBUF):
                    c = i * NBUF + b
                    b2 = b % 2
                    pltpu.make_async_copy(
                        table.at[hr_v.at[b]], pair_v.at[b], sem_g.at[b]).wait()

                    def wait_w(b2=b2, c=c):
                        pltpu.make_async_copy(
                            comp_t.at[b2], out_slice(c), sem_w.at[b2]).wait()
                    if b >= 2:
                        wait_w()
                    else:
                        pl.when(i > 0)(wait_w)
                    extract(b, b2)
                    pltpu.async_copy(comp_t.at[b2], out_slice(c), sem_w.at[b2])

                    @pl.when(c + NBUF < NCHUNK)
                    def _(b=b, c=c):
                        pltpu.async_copy(
                            idx_slice(c + NBUF), idx_v.at[b], sem_i.at[b])
                return carry

            lax.fori_loop(0, NCHUNK // NBUF, body, 0)
            for b2 in range(2):
                pltpu.make_async_copy(
                    comp_t.at[b2], out_slice(b2), sem_w.at[b2]).wait()


def kernel(values_0, values_1, values_2, values_3, W0, W1, W2, W3):
    wp = [w.reshape(w.shape[0] // 2, 2 * EMBED_DIM)
          for w in (W0, W1, W2, W3)]
    res = _grouped_embedding(values_0, values_1, values_2, values_3, *wp)
    return res.T


# speedup vs baseline: 1.5217x; 1.2116x over previous
import functools

import jax
import jax.numpy as jnp
from jax import lax
from jax.experimental import pallas as pl
from jax.experimental.pallas import tpu as pltpu
from jax.experimental.pallas import tpu_sc as plsc

EMBED_DIM = 64
VALUES_LEN = 81920
NUM_TABLES = 4
NUM_WORKERS = 32
WORKERS_PER_TABLE = NUM_WORKERS // NUM_TABLES
PER_W = VALUES_LEN // WORKERS_PER_TABLE
CHUNK = 128
NCHUNK = PER_W // CHUNK
NBUF = 8

_mesh = plsc.VectorSubcoreMesh(core_axis_name="c", subcore_axis_name="s")


@functools.partial(
    pl.kernel,
    mesh=_mesh,
    out_type=jax.ShapeDtypeStruct((NUM_TABLES * VALUES_LEN, EMBED_DIM),
                                  jnp.float32),
    scratch_types=[
        pltpu.VMEM((PER_W,), jnp.int32),
        pltpu.VMEM((NBUF, CHUNK, EMBED_DIM), jnp.float32),
        pltpu.SemaphoreType.DMA((NBUF,)),
        pltpu.SemaphoreType.DMA((NBUF,)),
    ],
    compiler_params=pltpu.CompilerParams(use_tc_tiling_on_sc=False),
)
def _grouped_embedding(v0, v1, v2, v3, w0, w1, w2, w3, out,
                       idx_v, rows_v, sem_g, sem_w):
    wid = lax.axis_index("s") * 2 + lax.axis_index("c")
    table_id = wid // WORKERS_PER_TABLE
    sub = wid % WORKERS_PER_TABLE

    for t, (vals, table) in enumerate(
        ((v0, w0), (v1, w1), (v2, w2), (v3, w3))):

        @pl.when(table_id == t)
        def _(vals=vals, table=table, t=t):
            out_base = t * VALUES_LEN + sub * PER_W
            pltpu.sync_copy(vals.at[pl.ds(sub * PER_W, PER_W)], idx_v)

            def out_slice(c):
                return out.at[pl.ds(out_base + c * CHUNK, CHUNK), :]

            def body(i, carry):
                gathers = []
                for b in range(NBUF):
                    c = i * NBUF + b

                    @pl.when(i > 0)
                    def _(b=b, c=c):
                        pltpu.make_async_copy(
                            rows_v.at[b], out_slice(c), sem_w.at[b]).wait()

                    gathers.append(pltpu.async_copy(
                        table.at[idx_v.at[pl.ds(c * CHUNK, CHUNK)]],
                        rows_v.at[b], sem_g.at[b]))
                for b in range(NBUF):
                    c = i * NBUF + b
                    gathers[b].wait()
                    pltpu.async_copy(rows_v.at[b], out_slice(c), sem_w.at[b])
                return carry

            lax.fori_loop(0, NCHUNK // NBUF, body, 0)
            for b in range(NBUF):
                pltpu.make_async_copy(
                    rows_v.at[b], out_slice(b), sem_w.at[b]).wait()


def kernel(values_0, values_1, values_2, values_3, W0, W1, W2, W3):
    return _grouped_embedding(values_0, values_1, values_2, values_3,
                              W0, W1, W2, W3)
